# Initial kernel scaffold; baseline (speedup 1.0000x reference)
import jax
import jax.numpy as jnp
from jax.experimental import pallas as pl


def kernel(p, x, o, Wq, bq, Wk, bk, Wv, bv, p1_W, p1_b, pbn_g, pbn_b, p2_W, p2_b, wbn1_g, wbn1_b, w1_W, w1_b, wbn2_g, wbn2_b, w2_W, w2_b):
    raise NotImplementedError("write your pallas kernel here")


# speedup vs baseline: 4.2958x; 4.2958x over previous
import functools

import jax
import jax.numpy as jnp
from jax import lax
from jax.experimental import pallas as pl
from jax.experimental.pallas import tpu as pltpu
from jax.experimental.pallas import tpu_sc as plsc

N = 10000
NP = 10240
IN = 128
MID = 128
OUT = 128
S = 8
NS = 16
CG = OUT // S
MTOT = float(N * NS)
EPS = 1e-5

def _pack_bf16(x):
    half = x.shape[-1] // 2
    hi = x[..., :half].astype(jnp.bfloat16).astype(jnp.float32)
    lo = x[..., half:].astype(jnp.bfloat16).astype(jnp.float32)
    hi_i = lax.bitcast_convert_type(hi, jnp.int32)
    lo_i = lax.bitcast_convert_type(lo, jnp.int32)
    return lax.bitcast_convert_type(
        hi_i | lax.shift_right_logical(lo_i, 16), jnp.float32)


def _unpack_bf16(xp):
    xi = lax.bitcast_convert_type(xp, jnp.int32)
    hi = lax.bitcast_convert_type(xi & jnp.int32(-65536), jnp.float32)
    lo = lax.bitcast_convert_type(lax.shift_left(xi, jnp.int32(16)),
                                  jnp.float32)
    return jnp.concatenate([hi, lo], axis=-1)


_RB1 = 1024


def _qkv_body(x_ref, wq_ref, bq_ref, wk_ref, bk_ref, wv_ref, bv_ref,
              xq_ref, kv_ref):
    x = x_ref[...]
    xq_ref[...] = jnp.dot(x, wq_ref[...], preferred_element_type=jnp.float32) + bq_ref[...]
    xk = jnp.dot(x, wk_ref[...], preferred_element_type=jnp.float32) + bk_ref[...]
    xv = jnp.dot(x, wv_ref[...], preferred_element_type=jnp.float32) + bv_ref[...]
    kv_ref[...] = jnp.concatenate([_pack_bf16(xk), _pack_bf16(xv)], axis=-1)


def _qkv(xpad, Wq, bq, Wk, bk, Wv, bv):
    full = pl.BlockSpec((128, 128), lambda i: (0, 0))
    bias = pl.BlockSpec((1, 128), lambda i: (0, 0))
    row = pl.BlockSpec((_RB1, 128), lambda i: (i, 0))
    return pl.pallas_call(
        _qkv_body,
        grid=(NP // _RB1,),
        in_specs=[row, full, bias, full, bias, full, bias],
        out_specs=[row, row],
        out_shape=[jax.ShapeDtypeStruct((NP, 128), jnp.float32),
                   jax.ShapeDtypeStruct((NP, 128), jnp.float32)],
    )(xpad, Wq, bq.reshape(1, 128), Wk, bk.reshape(1, 128), Wv, bv.reshape(1, 128))


_BQ = 256


def _knn_body(pq_ref, pT_ref, idx_ref):
    pq = pq_ref[...]
    pT = pT_ref[...]
    sqk = jnp.sum(pT * pT, axis=0, keepdims=True)
    k1 = lax.broadcasted_iota(jnp.int32, (1, NP), 1)
    sqk = jnp.where(k1 >= N, jnp.float32(jnp.inf), sqk)
    sqq = jnp.sum(pq * pq, axis=1, keepdims=True)
    d = sqq + sqk - 2.0 * jnp.dot(pq, pT, preferred_element_type=jnp.float32)
    kiof = lax.broadcasted_iota(jnp.int32, (_BQ, NP), 1).astype(jnp.float32)
    BIGF = jnp.float32(2.0**30)
    cols = []
    for _ in range(NS):
        m = jnp.min(d, axis=1, keepdims=True)
        idxf = jnp.min(jnp.where(d <= m, kiof, BIGF), axis=1,
                       keepdims=True)
        cols.append(idxf.astype(jnp.int32))
        d = jnp.where(kiof == idxf, jnp.float32(jnp.inf), d)
    idx_ref[...] = jnp.concatenate(cols, axis=1)


def _knn(pp, pT):
    return pl.pallas_call(
        _knn_body,
        grid=(NP // _BQ,),
        in_specs=[pl.BlockSpec((_BQ, 16), lambda i: (i, 0)),
                  pl.BlockSpec((16, NP), lambda i: (0, 0))],
        out_specs=pl.BlockSpec((_BQ, NS), lambda i: (i, 0)),
        out_shape=jax.ShapeDtypeStruct((NP, NS), jnp.int32),
    )(pp, pT)


_NWORK = 32
_BTOT = NP * NS
_PERW = _BTOT // _NWORK
_CHUNK = 128
_NCH = _PERW // _CHUNK


def _sc_gather_body(idx_hbm, kv_hbm, px_hbm, py_hbm, pz_hbm,
                    kvg_hbm, pg_hbm,
                    idx_v, bk_v, bp_v, px_v, py_v, pz_v, sk):
    wid = lax.axis_index("s") * 2 + lax.axis_index("c")
    base = wid * _PERW
    pltpu.sync_copy(px_hbm, px_v)
    pltpu.sync_copy(py_hbm, py_v)
    pltpu.sync_copy(pz_hbm, pz_v)
    zero16 = jnp.zeros((16,), jnp.float32)
    for r in range(_CHUNK):
        bp_v[r * 16:(r + 1) * 16] = zero16
    lane = lax.iota(jnp.int32, 16)

    def chunk(t, carry):
        off = base + t * _CHUNK
        pltpu.sync_copy(idx_hbm.at[pl.ds(off, _CHUNK)], idx_v)
        ck = pltpu.async_copy(kv_hbm.at[idx_v], bk_v, sk)
        for g in range(_CHUNK // 16):
            nbr = idx_v[g * 16:(g + 1) * 16]
            q = jnp.full((16,), off // 16 + g, jnp.int32)
            for c, pc_v in enumerate((px_v, py_v, pz_v)):
                rel = plsc.load_gather(pc_v, [nbr]) - plsc.load_gather(pc_v, [q])
                plsc.store_scatter(bp_v, [lane * 16 + (g * 256 + c)], rel)
        ck.wait()
        pltpu.sync_copy(bk_v, kvg_hbm.at[pl.ds(off, _CHUNK)])
        pltpu.sync_copy(bp_v, pg_hbm.at[pl.ds(off * 16, _CHUNK * 16)])
        return carry

    lax.fori_loop(0, _NCH, chunk, 0)


def _sc_gather(idxf, kv, px, py, pz):
    mesh = plsc.VectorSubcoreMesh(core_axis_name="c", subcore_axis_name="s")
    call = pl.kernel(
        _sc_gather_body,
        out_type=[jax.ShapeDtypeStruct((_BTOT, 128), jnp.float32),
                  jax.ShapeDtypeStruct((_BTOT * 16,), jnp.float32)],
        mesh=mesh,
        compiler_params=pltpu.CompilerParams(needs_layout_passes=False),
        scratch_types=[pltpu.VMEM((_CHUNK,), jnp.int32),
                       pltpu.VMEM((_CHUNK, 128), jnp.float32),
                       pltpu.VMEM((_CHUNK * 16,), jnp.float32),
                       pltpu.VMEM((NP,), jnp.float32),
                       pltpu.VMEM((NP,), jnp.float32),
                       pltpu.VMEM((NP,), jnp.float32),
                       pltpu.SemaphoreType.DMA],
    )
    return call(idxf, kv, px, py, pz)


_BP = 64
_RB4 = _BP * NS


def _bn_apply(t, s, g, b):
    mean = s[0:1, :] / MTOT
    var = s[1:2, :] / MTOT - mean * mean
    return (t - mean) * lax.rsqrt(var + EPS) * g + b


def _pos_enc(pg_ref, sA_ref, p1W_ref, p1b_ref, pbng_ref, pbnb_ref,
             p2W_ref, p2b_ref):
    pr_ = pg_ref[...].reshape(_RB4, 16)
    pr1 = jnp.dot(pr_, p1W_ref[...], preferred_element_type=jnp.float32) + p1b_ref[...]
    pr1 = jnp.maximum(_bn_apply(pr1, sA_ref[...], pbng_ref[...], pbnb_ref[...]), 0.0)
    return jnp.dot(pr1, p2W_ref[...], preferred_element_type=jnp.float32) + p2b_ref[...]


def _rowmask(i):
    rio = lax.broadcasted_iota(jnp.int32, (_RB4, 1), 0) + i * _RB4
    return jnp.where(rio < N * NS, jnp.float32(1.0), jnp.float32(0.0))


def _accum_stats(outs_ref, t, msk, i, width):
    s1 = jnp.sum(t * msk, axis=0, keepdims=True)
    s2 = jnp.sum(t * t * msk, axis=0, keepdims=True)
    acc = jnp.concatenate([s1, s2, jnp.zeros((6, width), jnp.float32)], axis=0)

    @pl.when(i == 0)
    def _():
        outs_ref[...] = jnp.zeros_like(outs_ref)

    outs_ref[...] += acc


def _prestats_body(pg_ref, p1W_ref, p1b_ref, outs_ref):
    i = pl.program_id(0)
    pr_ = pg_ref[...].reshape(_RB4, 16)
    pr1 = jnp.dot(pr_, p1W_ref[...], preferred_element_type=jnp.float32) + p1b_ref[...]
    _accum_stats(outs_ref, pr1, _rowmask(i), i, 16)


def _w0stats_body(xkg_ref, pg_ref, xq_ref, sA_ref,
                  p1W_ref, p1b_ref, pbng_ref, pbnb_ref, p2W_ref, p2b_ref,
                  outs_ref):
    i = pl.program_id(0)
    pr = _pos_enc(pg_ref, sA_ref, p1W_ref, p1b_ref, pbng_ref,
                  pbnb_ref, p2W_ref, p2b_ref)
    xqe = jnp.broadcast_to(xq_ref[...][:, None, :], (_BP, NS, IN)).reshape(_RB4, IN)
    w0 = _unpack_bf16(xkg_ref[...].reshape(_RB4, IN)[:, :IN // 2]) - xqe + pr
    _accum_stats(outs_ref, w0, _rowmask(i), i, 128)


def _w1_body(xkg_ref, pg_ref, xq_ref, sA_ref, sB_ref,
             p1W_ref, p1b_ref, pbng_ref, pbnb_ref, p2W_ref, p2b_ref,
             wbn1g_ref, wbn1b_ref, w1W_ref, w1b_ref,
             w1g_ref, outs_ref):
    i = pl.program_id(0)
    pr = _pos_enc(pg_ref, sA_ref, p1W_ref, p1b_ref, pbng_ref,
                  pbnb_ref, p2W_ref, p2b_ref)
    xqe = jnp.broadcast_to(xq_ref[...][:, None, :], (_BP, NS, IN)).reshape(_RB4, IN)
    w0 = _unpack_bf16(xkg_ref[...].reshape(_RB4, IN)[:, :IN // 2]) - xqe + pr
    w0n = jnp.maximum(_bn_apply(w0, sB_ref[...], wbn1g_ref[...], wbn1b_ref[...]), 0.0)
    w1 = jnp.dot(w0n, w1W_ref[...], preferred_element_type=jnp.float32) + w1b_ref[...]
    w1g_ref[...] = w1.reshape(_BP, NS, CG)
    _accum_stats(outs_ref, w1, _rowmask(i), i, CG)


def _final_body(w1g_ref, xvg_ref, pg_ref, sA_ref, sC_ref,
                p1W_ref, p1b_ref, pbng_ref, pbnb_ref, p2W_ref, p2b_ref,
                wbn2g_ref, wbn2b_ref, w2W_ref, w2b_ref,
                out_ref):
    pr = _pos_enc(pg_ref, sA_ref, p1W_ref, p1b_ref, pbng_ref,
                  pbnb_ref, p2W_ref, p2b_ref)
    w1 = w1g_ref[...].reshape(_RB4, CG)
    w1n = jnp.maximum(_bn_apply(w1, sC_ref[...], wbn2g_ref[...], wbn2b_ref[...]), 0.0)
    w2 = (jnp.dot(w1n, w2W_ref[...], preferred_element_type=jnp.float32)
          + w2b_ref[...]).reshape(_BP, NS, CG)
    mx = jnp.max(w2, axis=1, keepdims=True)
    e = jnp.exp(w2 - mx)
    att = e / jnp.sum(e, axis=1, keepdims=True)
    attt = jnp.concatenate([att] * S, axis=2)
    xv_u = _unpack_bf16(xvg_ref[...].reshape(_RB4, OUT)[:, OUT // 2:])
    v = xv_u.reshape(_BP, NS, OUT) + pr.reshape(_BP, NS, OUT)
    out_ref[...] = jnp.sum(v * attt, axis=1)


def _blk3(c):
    return pl.BlockSpec((_BP, NS, c), lambda i: (i, 0, 0))


def _blk2(c):
    return pl.BlockSpec((_BP, c), lambda i: (i, 0))


def _cst(r, c):
    return pl.BlockSpec((r, c), lambda i: (0, 0))


_G4 = (NP // _BP,)


def _prestats(pg3, p1W16, p1b16):
    return pl.pallas_call(
        _prestats_body,
        grid=_G4,
        in_specs=[_blk3(16), _cst(16, 16), _cst(1, 16)],
        out_specs=_cst(8, 16),
        out_shape=jax.ShapeDtypeStruct((8, 16), jnp.float32),
    )(pg3, p1W16, p1b16)


def _w0stats(xkg3, pg3, xq, sA, p1W16, p1b16, pbng16, pbnb16, p2W16, p2b):
    return pl.pallas_call(
        _w0stats_body,
        grid=_G4,
        in_specs=[_blk3(128), _blk3(16), _blk2(128), _cst(8, 16),
                  _cst(16, 16), _cst(1, 16), _cst(1, 16), _cst(1, 16),
                  _cst(16, 128), _cst(1, 128)],
        out_specs=_cst(8, 128),
        out_shape=jax.ShapeDtypeStruct((8, 128), jnp.float32),
    )(xkg3, pg3, xq, sA, p1W16, p1b16, pbng16, pbnb16, p2W16, p2b)


def _w1pass(xkg3, pg3, xq, sA, sB, p1W16, p1b16, pbng16, pbnb16, p2W16,
            p2b, wbn1g, wbn1b, w1W, w1b):
    return pl.pallas_call(
        _w1_body,
        grid=_G4,
        in_specs=[_blk3(128), _blk3(16), _blk2(128), _cst(8, 16),
                  _cst(8, 128), _cst(16, 16), _cst(1, 16), _cst(1, 16),
                  _cst(1, 16), _cst(16, 128), _cst(1, 128), _cst(1, 128),
                  _cst(1, 128), _cst(128, 16), _cst(1, 16)],
        out_specs=[_blk3(CG), _cst(8, CG)],
        out_shape=[jax.ShapeDtypeStruct((NP, NS, CG), jnp.float32),
                   jax.ShapeDtypeStruct((8, CG), jnp.float32)],
    )(xkg3, pg3, xq, sA, sB, p1W16, p1b16, pbng16, pbnb16, p2W16, p2b,
      wbn1g, wbn1b, w1W, w1b)


def _finalpass(w1g3, xvg3, pg3, sA, sC, p1W16, p1b16, pbng16, pbnb16,
               p2W16, p2b, wbn2g, wbn2b, w2W, w2b):
    return pl.pallas_call(
        _final_body,
        grid=_G4,
        in_specs=[_blk3(CG), _blk3(128), _blk3(16), _cst(8, 16),
                  _cst(8, CG), _cst(16, 16), _cst(1, 16), _cst(1, 16),
                  _cst(1, 16), _cst(16, 128), _cst(1, 128), _cst(1, 16),
                  _cst(1, 16), _cst(16, 16), _cst(1, 16)],
        out_specs=_blk2(128),
        out_shape=jax.ShapeDtypeStruct((NP, OUT), jnp.float32),
    )(w1g3, xvg3, pg3, sA, sC, p1W16, p1b16, pbng16, pbnb16, p2W16, p2b,
      wbn2g, wbn2b, w2W, w2b)


def kernel(p, x, o, Wq, bq, Wk, bk, Wv, bv, p1_W, p1_b, pbn_g, pbn_b, p2_W,
           p2_b, wbn1_g, wbn1_b, w1_W, w1_b, wbn2_g, wbn2_b, w2_W, w2_b):
    f32 = jnp.float32
    p = p.astype(f32)
    x = x.astype(f32)
    pp = jnp.zeros((NP, 16), f32).at[:N, :3].set(p)
    pT = jnp.zeros((16, NP), f32).at[:3, :N].set(p.T)
    xpad = jnp.zeros((NP, IN), f32).at[:N].set(x)

    p1W16 = jnp.zeros((16, 16), f32).at[:3, :3].set(p1_W)
    p1b16 = jnp.zeros((1, 16), f32).at[0, :3].set(p1_b)
    pbng16 = jnp.ones((1, 16), f32).at[0, :3].set(pbn_g)
    pbnb16 = jnp.zeros((1, 16), f32).at[0, :3].set(pbn_b)
    p2W16 = jnp.zeros((16, OUT), f32).at[:3, :].set(p2_W)
    p2b = p2_b.reshape(1, OUT)
    wbn1g = wbn1_g.reshape(1, MID)
    wbn1b = wbn1_b.reshape(1, MID)
    w1b = w1_b.reshape(1, CG)
    wbn2g = wbn2_g.reshape(1, CG)
    wbn2b = wbn2_b.reshape(1, CG)
    w2b = w2_b.reshape(1, CG)

    px = jnp.zeros((NP,), f32).at[:N].set(p[:, 0])
    py = jnp.zeros((NP,), f32).at[:N].set(p[:, 1])
    pz = jnp.zeros((NP,), f32).at[:N].set(p[:, 2])

    xq, kv = _qkv(xpad, Wq, bq, Wk, bk, Wv, bv)
    idx = _knn(pp, pT)
    idxf = idx.reshape(_BTOT)
    kvg, pg = _sc_gather(idxf, kv, px, py, pz)
    xkg3 = kvg.reshape(NP, NS, 128)
    xvg3 = xkg3
    pg3 = pg.reshape(NP, NS, 16)

    sA = _prestats(pg3, p1W16, p1b16)
    sB = _w0stats(xkg3, pg3, xq, sA, p1W16, p1b16, pbng16, pbnb16,
                  p2W16, p2b)
    w1g3, sC = _w1pass(xkg3, pg3, xq, sA, sB, p1W16, p1b16, pbng16,
                       pbnb16, p2W16, p2b, wbn1g, wbn1b, w1_W, w1b)
    outp = _finalpass(w1g3, xvg3, pg3, sA, sC, p1W16, p1b16, pbng16,
                      pbnb16, p2W16, p2b, wbn2g, wbn2b, w2_W, w2b)
    return outp[:N]
